# baseline (speedup 1.0000x reference)
import functools

import jax
import jax.numpy as jnp
from jax import lax
from jax.experimental import pallas as pl
from jax.experimental.pallas import tpu as pltpu
from jax.experimental.pallas import tpu_sc as plsc

VOCAB = 1000000
D = 64
B = 16384
L = 50
ROWS = B * L
NC = 2
NS = 16
NW = NC * NS
BPW = B // NW
NE = 16
NCHUNK = BPW // NE
NBUF = 2
CROWS = NE * L


PBLK = 6400


def _pack_kernel(xt_ref, o_ref):
    x = xt_ref[:].T
    x3 = x.reshape(PBLK // 2, 2, D)
    o_ref[:] = jnp.concatenate([x3[:, 0, :], x3[:, 1, :]], axis=1)


def _tc_pack(table_t):
    return pl.pallas_call(
        _pack_kernel,
        grid=(pl.cdiv(VOCAB, PBLK),),
        in_specs=[pl.BlockSpec((D, PBLK), lambda i: (0, i))],
        out_specs=pl.BlockSpec((PBLK // 2, 2 * D), lambda i: (i, 0)),
        out_shape=jax.ShapeDtypeStruct((VOCAB // 2, 2 * D), jnp.float32),
    )(table_t)


HB = B // 2
HPW = HB // NW
HCHUNK = HPW // NE


def _gather_body(eoff, idx_hbm, table_hbm, out_hbm, idx_v, slab0, slab1,
                 sem0, sem1):
    wid = lax.axis_index("s") * NC + lax.axis_index("c")
    ebase = eoff + wid * HPW
    rbase = wid * (HPW * L)
    slabs = (slab0, slab1)
    sems = (sem0, sem1)

    pltpu.sync_copy(idx_hbm.at[pl.ds(ebase, HPW)], idx_v)

    def fire(buf, chunk):
        for j in range(NE):
            pltpu.async_copy(
                table_hbm.at[idx_v.at[chunk * NE + j]],
                slabs[buf].at[pl.ds(j * L, L)],
                sems[buf],
            )

    for buf in range(NBUF):
        fire(buf, buf)

    def step(it, carry):
        chunk0 = it * NBUF
        for buf in range(NBUF):
            chunk = chunk0 + buf
            pltpu.make_async_copy(
                table_hbm.at[pl.ds(0, CROWS)], slabs[buf], sems[buf]
            ).wait()
            pltpu.sync_copy(
                slabs[buf], out_hbm.at[pl.ds(rbase + chunk * CROWS, CROWS)]
            )
            nxt = chunk + NBUF

            @pl.when(nxt < HCHUNK)
            def _():
                fire(buf, nxt)

        return carry

    lax.fori_loop(0, HCHUNK // NBUF, step, 0)


def _sc_gather(idxs, table, eoff):
    mesh = plsc.VectorSubcoreMesh(core_axis_name="c", subcore_axis_name="s")
    return pl.kernel(
        functools.partial(_gather_body, eoff),
        mesh=mesh,
        out_type=jax.ShapeDtypeStruct((HB * L, D), jnp.float32),
        scratch_types=[
            pltpu.VMEM((HPW, L), jnp.int32),
            pltpu.VMEM((CROWS, D), jnp.float32),
            pltpu.VMEM((CROWS, D), jnp.float32),
            pltpu.SemaphoreType.DMA,
            pltpu.SemaphoreType.DMA,
        ],
        compiler_params=pltpu.CompilerParams(use_tc_tiling_on_sc=False),
    )(idxs, table)


BBt = 128


def _linear_kernel(x_ref, w4_ref, b4_ref, o_ref):
    y = (
        jnp.dot(x_ref[:], w4_ref[:], preferred_element_type=jnp.float32)
        + b4_ref[:]
    )
    y2 = y.reshape(BBt, L * D)
    o_ref[:] = y2.T


def _tc_linear_a(embs2, w4, b4):
    return pl.pallas_call(
        _linear_kernel,
        grid=(HB // BBt,),
        in_specs=[
            pl.BlockSpec((BBt * L // 2, 2 * D), lambda i: (i, 0)),
            pl.BlockSpec((2 * D, 2 * D), lambda i: (0, 0)),
            pl.BlockSpec((1, 2 * D), lambda i: (0, 0)),
        ],
        out_specs=pl.BlockSpec((L * D, BBt), lambda i: (0, i)),
        out_shape=jax.ShapeDtypeStruct((L * D, B), jnp.float32),
    )(embs2, w4, b4)


def _linear_kernel_b(prev_ref, x_ref, w4_ref, b4_ref, o_ref):
    del prev_ref
    _linear_kernel(x_ref, w4_ref, b4_ref, o_ref)


def _tc_linear_b(prev, embs2, w4, b4):
    return pl.pallas_call(
        _linear_kernel_b,
        grid=(HB // BBt,),
        in_specs=[
            pl.BlockSpec(memory_space=pl.ANY),
            pl.BlockSpec((BBt * L // 2, 2 * D), lambda i: (i, 0)),
            pl.BlockSpec((2 * D, 2 * D), lambda i: (0, 0)),
            pl.BlockSpec((1, 2 * D), lambda i: (0, 0)),
        ],
        out_specs=pl.BlockSpec((L * D, BBt), lambda i: (0, i + HB // BBt)),
        out_shape=jax.ShapeDtypeStruct((L * D, B), jnp.float32),
        input_output_aliases={0: 0},
    )(prev, embs2, w4, b4)


def kernel(idxs, table, W, b):
    t2 = _tc_pack(table.T)
    tt = t2.reshape(VOCAB, D)
    embs_a = _sc_gather(idxs, tt, 0)
    embs_b = _sc_gather(idxs, tt, HB)
    wt = W.T
    w4 = jnp.zeros((2 * D, 2 * D), jnp.float32)
    w4 = w4.at[:D, :D].set(wt).at[D:, D:].set(wt)
    b4 = jnp.concatenate([b, b]).reshape(1, 2 * D)
    out_a = _tc_linear_a(embs_a.reshape(HB * L // 2, 2 * D), w4, b4)
    out_t = _tc_linear_b(out_a, embs_b.reshape(HB * L // 2, 2 * D), w4, b4)
    return out_t.reshape(L, D, B).transpose(2, 0, 1)


# speedup vs baseline: 1.0369x; 1.0369x over previous
import functools

import jax
import jax.numpy as jnp
from jax import lax
from jax.experimental import pallas as pl
from jax.experimental.pallas import tpu as pltpu
from jax.experimental.pallas import tpu_sc as plsc

VOCAB = 1000000
D = 64
B = 16384
L = 50
ROWS = B * L
NC = 2
NS = 16
NW = NC * NS
BPW = B // NW
NE = 16
NCHUNK = BPW // NE
NBUF = 2
CROWS = NE * L


PBLK = 6400


def _pack_kernel(xt_ref, o_ref):
    x = xt_ref[:].T
    x3 = x.reshape(PBLK // 2, 2, D)
    o_ref[:] = jnp.concatenate([x3[:, 0, :], x3[:, 1, :]], axis=1)


def _tc_pack(table_t):
    return pl.pallas_call(
        _pack_kernel,
        grid=(pl.cdiv(VOCAB, PBLK),),
        in_specs=[pl.BlockSpec((D, PBLK), lambda i: (0, i))],
        out_specs=pl.BlockSpec((PBLK // 2, 2 * D), lambda i: (i, 0)),
        out_shape=jax.ShapeDtypeStruct((VOCAB // 2, 2 * D), jnp.float32),
    )(table_t)


HB = B // 4
HPW = HB // NW
HCHUNK = HPW // NE


def _gather_body(idx_hbm, table_hbm, out_hbm, idx_v, slab0, slab1, sem0, sem1):
    wid = lax.axis_index("s") * NC + lax.axis_index("c")
    ebase = wid * HPW
    rbase = wid * (HPW * L)
    slabs = (slab0, slab1)
    sems = (sem0, sem1)

    pltpu.sync_copy(idx_hbm.at[pl.ds(ebase, HPW)], idx_v)

    def fire(buf, chunk):
        for j in range(NE):
            pltpu.async_copy(
                table_hbm.at[idx_v.at[chunk * NE + j]],
                slabs[buf].at[pl.ds(j * L, L)],
                sems[buf],
            )

    for buf in range(NBUF):
        fire(buf, buf)

    def step(it, carry):
        chunk0 = it * NBUF
        for buf in range(NBUF):
            chunk = chunk0 + buf
            pltpu.make_async_copy(
                table_hbm.at[pl.ds(0, CROWS)], slabs[buf], sems[buf]
            ).wait()
            pltpu.sync_copy(
                slabs[buf], out_hbm.at[pl.ds(rbase + chunk * CROWS, CROWS)]
            )
            nxt = chunk + NBUF

            @pl.when(nxt < HCHUNK)
            def _():
                fire(buf, nxt)

        return carry

    lax.fori_loop(0, HCHUNK // NBUF, step, 0)


def _sc_gather(idxs, table):
    mesh = plsc.VectorSubcoreMesh(core_axis_name="c", subcore_axis_name="s")
    return pl.kernel(
        _gather_body,
        mesh=mesh,
        out_type=jax.ShapeDtypeStruct((HB * L, D), jnp.float32),
        scratch_types=[
            pltpu.VMEM((HPW, L), jnp.int32),
            pltpu.VMEM((CROWS, D), jnp.float32),
            pltpu.VMEM((CROWS, D), jnp.float32),
            pltpu.SemaphoreType.DMA,
            pltpu.SemaphoreType.DMA,
        ],
        compiler_params=pltpu.CompilerParams(use_tc_tiling_on_sc=False),
    )(idxs, table)


BBt = 128


def _linear_kernel(x_ref, w4_ref, b4_ref, o_ref):
    y = (
        jnp.dot(x_ref[:], w4_ref[:], preferred_element_type=jnp.float32)
        + b4_ref[:]
    )
    y2 = y.reshape(BBt, L * D)
    o_ref[:] = y2.T


def _tc_linear_a(embs2, w4, b4):
    return pl.pallas_call(
        _linear_kernel,
        grid=(HB // BBt,),
        in_specs=[
            pl.BlockSpec((BBt * L // 2, 2 * D), lambda i: (i, 0)),
            pl.BlockSpec((2 * D, 2 * D), lambda i: (0, 0)),
            pl.BlockSpec((1, 2 * D), lambda i: (0, 0)),
        ],
        out_specs=pl.BlockSpec((L * D, BBt), lambda i: (0, i)),
        out_shape=jax.ShapeDtypeStruct((L * D, B), jnp.float32),
    )(embs2, w4, b4)


def _linear_kernel_b(prev_ref, x_ref, w4_ref, b4_ref, o_ref):
    del prev_ref
    _linear_kernel(x_ref, w4_ref, b4_ref, o_ref)


def _tc_linear_b(prev, embs2, w4, b4, part):
    off = part * (HB // BBt)
    return pl.pallas_call(
        _linear_kernel_b,
        grid=(HB // BBt,),
        in_specs=[
            pl.BlockSpec(memory_space=pl.ANY),
            pl.BlockSpec((BBt * L // 2, 2 * D), lambda i: (i, 0)),
            pl.BlockSpec((2 * D, 2 * D), lambda i: (0, 0)),
            pl.BlockSpec((1, 2 * D), lambda i: (0, 0)),
        ],
        out_specs=pl.BlockSpec((L * D, BBt), lambda i, _o=off: (0, i + _o)),
        out_shape=jax.ShapeDtypeStruct((L * D, B), jnp.float32),
        input_output_aliases={0: 0},
    )(prev, embs2, w4, b4)


def kernel(idxs, table, W, b):
    t2 = _tc_pack(table.T)
    tt = t2.reshape(VOCAB, D)
    parts = [_sc_gather(idxs[k * HB:(k + 1) * HB], tt) for k in range(4)]
    wt = W.T
    w4 = jnp.zeros((2 * D, 2 * D), jnp.float32)
    w4 = w4.at[:D, :D].set(wt).at[D:, D:].set(wt)
    b4 = jnp.concatenate([b, b]).reshape(1, 2 * D)
    out_t = _tc_linear_a(parts[0].reshape(HB * L // 2, 2 * D), w4, b4)
    for k in range(1, 4):
        out_t = _tc_linear_b(
            out_t, parts[k].reshape(HB * L // 2, 2 * D), w4, b4, k
        )
    return out_t.reshape(L, D, B).transpose(2, 0, 1)
